# baseline (speedup 1.0000x reference)
import functools

import jax
import jax.numpy as jnp
import numpy as np
from jax import lax
from jax.experimental import pallas as pl
from jax.experimental.pallas import tpu as pltpu
from jax.experimental.pallas import tpu_sc as plsc

B, L, M, LOCAL_DIM = 16384, 100, 64, 2
GW = 7
G = (L + GW - 1) // GW
E = 1 << GW
TAB_ROWS = (G - 1) * E + 8
TB = 16384

NC, NS, LANES = 2, 16, 16
NW = NC * NS
BPW = B // NW


def _pack_weights() -> np.ndarray:
    w = np.zeros((G, L), np.float32)
    for l in range(L):
        w[l // GW, l] = float(1 << (l % GW))
    return w


def _prep_body(x_ref, w_ref, eps_ref, pre_ref, tab_ref):
    x = x_ref[...].astype(jnp.float32)
    codes = lax.dot_general(w_ref[...], x, (((1,), (0,)), ((), ())),
                            preferred_element_type=jnp.float32)
    gid = lax.broadcasted_iota(jnp.int32, codes.shape, 0)
    pre_ref[...] = (codes.astype(jnp.int32) + gid * E) * M

    @pl.when(pl.program_id(0) == 0)
    def _build_tables():
        e0t = eps_ref[0].T
        e1t = eps_ref[1].T
        for g in range(G):
            width = min(GW, L - g * GW)
            rows = E if width == GW else 8
            pbits = lax.broadcasted_iota(jnp.int32, (rows, 1), 0)
            t = jnp.ones((rows, M), jnp.float32)
            for j in range(width):
                site = g * GW + j
                e0 = e0t[site]
                de = e1t[site] - e0
                bit = ((pbits >> j) & 1).astype(jnp.float32)
                t = t * (e0[None, :] + bit * de[None, :])
            tab_ref[pl.ds(g * E, rows)] = t


_prep = pl.pallas_call(
    _prep_body,
    grid=(B // TB,),
    in_specs=[
        pl.BlockSpec((L, TB), lambda i: (0, i)),
        pl.BlockSpec((G, L), lambda i: (0, 0)),
        pl.BlockSpec((LOCAL_DIM, M, L), lambda i: (0, 0, 0)),
    ],
    out_specs=[
        pl.BlockSpec((G, TB), lambda i: (0, i)),
        pl.BlockSpec((TAB_ROWS, M), lambda i: (0, 0)),
    ],
    out_shape=[
        jax.ShapeDtypeStruct((G, B), jnp.int32),
        jax.ShapeDtypeStruct((TAB_ROWS, M), jnp.float32),
    ],
)


def _sc_body(tab_hbm, pre_hbm, out_hbm, tab_v, codes_v, out_v):
    wid = lax.axis_index("s") * NC + lax.axis_index("c")
    base = wid * BPW
    pltpu.sync_copy(tab_hbm, tab_v)
    pltpu.sync_copy(pre_hbm.at[:, pl.ds(base, BPW)], codes_v)

    NK = M // LANES
    lane_iota = lax.iota(jnp.int32, LANES)

    @plsc.parallel_loop(0, BPW // LANES, 1, unroll=1)
    def chunk_body(c):
        kv = [codes_v[g, pl.ds(c * LANES, LANES)] for g in range(G)]
        merged = jnp.zeros((LANES,), jnp.float32)
        for i in range(LANES):
            bases = [kv[g][i] for g in range(G)]
            tots = []
            for kp in range(NK // 2):
                pair = []
                for k in (2 * kp, 2 * kp + 1):
                    acc = tab_v[pl.ds(bases[0] + k * LANES, LANES)]
                    for g in range(1, G):
                        acc = acc * tab_v[pl.ds(bases[g] + k * LANES, LANES)]
                    pair.append(acc)
                tots.append(lax.reduce_sum_p.bind(pair[0] + pair[1], axes=(0,)))
            s = tots[0] + tots[1]
            merged = jnp.where(lane_iota == i, s, merged)
        out_v[pl.ds(c * LANES, LANES)] = merged
    pltpu.sync_copy(out_v, out_hbm.at[pl.ds(base, BPW)])


@functools.cache
def _sc_lookup():
    return pl.kernel(
        _sc_body,
        out_type=jax.ShapeDtypeStruct((B,), jnp.float32),
        mesh=plsc.VectorSubcoreMesh(core_axis_name="c", subcore_axis_name="s"),
        compiler_params=pltpu.CompilerParams(needs_layout_passes=False),
        scratch_types=[
            pltpu.VMEM((TAB_ROWS * M,), jnp.float32),
            pltpu.VMEM((G, BPW), jnp.int32),
            pltpu.VMEM((BPW,), jnp.float32),
        ],
    )


def kernel(inputs, epsilon):
    x = inputs
    if x.ndim == 1:
        x = jnp.expand_dims(x, 0)
    w = jnp.asarray(_pack_weights())
    pre, tab = _prep(x.T, w, epsilon)
    return _sc_lookup()(tab.reshape(-1), pre)


# speedup vs baseline: 1.0173x; 1.0173x over previous
import functools

import jax
import jax.numpy as jnp
import numpy as np
from jax import lax
from jax.experimental import pallas as pl
from jax.experimental.pallas import tpu as pltpu
from jax.experimental.pallas import tpu_sc as plsc

B, L, M, LOCAL_DIM = 16384, 100, 64, 2
GW = 7
G = (L + GW - 1) // GW
E = 1 << GW
TAB_ROWS = (G - 1) * E + 8
TB = 8192

NC, NS, LANES = 2, 16, 16
NW = NC * NS
BPW = B // NW


def _pack_weights() -> np.ndarray:
    w = np.zeros((G, L), np.float32)
    for l in range(L):
        w[l // GW, l] = float(1 << (l % GW))
    return w


def _prep_body(x_ref, w_ref, eps_ref, pre_ref, tab_ref):
    x = x_ref[...].astype(jnp.float32)
    codes = lax.dot_general(w_ref[...], x, (((1,), (0,)), ((), ())),
                            preferred_element_type=jnp.float32)
    gid = lax.broadcasted_iota(jnp.int32, codes.shape, 0)
    pre_ref[...] = (codes.astype(jnp.int32) + gid * E) * M

    @pl.when(pl.program_id(0) == 0)
    def _build_tables():
        e0t = eps_ref[0].T
        e1t = eps_ref[1].T
        for g in range(G):
            width = min(GW, L - g * GW)
            rows = E if width == GW else 8
            pbits = lax.broadcasted_iota(jnp.int32, (rows, 1), 0)
            t = jnp.ones((rows, M), jnp.float32)
            for j in range(width):
                site = g * GW + j
                e0 = e0t[site]
                de = e1t[site] - e0
                bit = ((pbits >> j) & 1).astype(jnp.float32)
                t = t * (e0[None, :] + bit * de[None, :])
            tab_ref[pl.ds(g * E, rows)] = t


_prep = pl.pallas_call(
    _prep_body,
    grid=(B // TB,),
    in_specs=[
        pl.BlockSpec((L, TB), lambda i: (0, i)),
        pl.BlockSpec((G, L), lambda i: (0, 0)),
        pl.BlockSpec((LOCAL_DIM, M, L), lambda i: (0, 0, 0)),
    ],
    out_specs=[
        pl.BlockSpec((G, TB), lambda i: (0, i)),
        pl.BlockSpec((TAB_ROWS, M), lambda i: (0, 0)),
    ],
    out_shape=[
        jax.ShapeDtypeStruct((G, B), jnp.int32),
        jax.ShapeDtypeStruct((TAB_ROWS, M), jnp.float32),
    ],
)


def _sc_body(tab_hbm, pre_hbm, out_hbm, tab_v, codes_v, out_v, dma_sem):
    wid = lax.axis_index("s") * NC + lax.axis_index("c")
    base = wid * BPW
    tab_cp = pltpu.async_copy(tab_hbm, tab_v, dma_sem)
    pltpu.sync_copy(pre_hbm.at[:, pl.ds(base, BPW)], codes_v)
    tab_cp.wait()

    NK = M // LANES
    lane_iota = lax.iota(jnp.int32, LANES)

    @plsc.parallel_loop(0, BPW // LANES, 1, unroll=1)
    def chunk_body(c):
        kv = [codes_v[g, pl.ds(c * LANES, LANES)] for g in range(G)]
        merged = jnp.zeros((LANES,), jnp.float32)
        for i in range(LANES):
            bases = [kv[g][i] for g in range(G)]
            tots = []
            for kp in range(NK // 2):
                pair = []
                for k in (2 * kp, 2 * kp + 1):
                    acc = tab_v[pl.ds(bases[0] + k * LANES, LANES)]
                    for g in range(1, G):
                        acc = acc * tab_v[pl.ds(bases[g] + k * LANES, LANES)]
                    pair.append(acc)
                tots.append(lax.reduce_sum_p.bind(pair[0] + pair[1], axes=(0,)))
            s = tots[0] + tots[1]
            merged = jnp.where(lane_iota == i, s, merged)
        out_v[pl.ds(c * LANES, LANES)] = merged
    pltpu.sync_copy(out_v, out_hbm.at[pl.ds(base, BPW)])


@functools.cache
def _sc_lookup():
    return pl.kernel(
        _sc_body,
        out_type=jax.ShapeDtypeStruct((B,), jnp.float32),
        mesh=plsc.VectorSubcoreMesh(core_axis_name="c", subcore_axis_name="s"),
        compiler_params=pltpu.CompilerParams(needs_layout_passes=False),
        scratch_types=[
            pltpu.VMEM((TAB_ROWS * M,), jnp.float32),
            pltpu.VMEM((G, BPW), jnp.int32),
            pltpu.VMEM((BPW,), jnp.float32),
            pltpu.SemaphoreType.DMA,
        ],
    )


def kernel(inputs, epsilon):
    x = inputs
    if x.ndim == 1:
        x = jnp.expand_dims(x, 0)
    w = jnp.asarray(_pack_weights())
    pre, tab = _prep(x.T, w, epsilon)
    return _sc_lookup()(tab.reshape(-1), pre)
